# baseline (speedup 1.0000x reference)
import jax
import jax.numpy as jnp
from jax import lax
from jax.experimental import pallas as pl
from jax.experimental.pallas import tpu as pltpu
from jax.experimental.pallas import tpu_sc as plsc

NC = 2
NS = 16
L = 16
NW = NC * NS

B = 16384
D = 16
S = 8
E = B // NW
C = 16
NCH = E // C
R = 512


def _sc_body(u_hbm, v_hbm,
             usr_hbm, item_hbm, rel_hbm,
             nuu_f, nui_f, niu_f, nii_f, nir_f,
             st_f, ratui_f, votui_f, timui_f,
             ratiu_f, votiu_f, timiu_f,
             uu_out, ui_out, iu_out, ii_out, ir_out,
             uself_out, vself_out,
             st_out, ratui_out, votui_out, timui_out,
             ratiu_out, votiu_out, timiu_out,
             u_v, v_v, fxu, fxv,
             iuu, iui, iiu, iii, iir,
             est, eratui, evotui, etimui, eratiu, evotiu, etimiu,
             uself_v, vself_v,
             ruu, rui, riu, rii, rir,
             sem1, sem2, semo):
    wid = lax.axis_index("s") * NC + lax.axis_index("c")
    base = pl.multiple_of(wid * E, E)
    iota = lax.iota(jnp.int32, L)

    pltpu.sync_copy(u_hbm.at[pl.ds(base, E)], u_v)
    pltpu.sync_copy(v_hbm.at[pl.ds(base, E)], v_v)

    def chunk_body(c, _):
        loc = pl.multiple_of(c * C, C)
        off = pl.multiple_of(base + c * C, C)
        uc = u_v.at[pl.ds(loc, C)]
        vc = v_v.at[pl.ds(loc, C)]

        ucv = u_v[pl.ds(loc, C)]
        vcv = v_v[pl.ds(loc, C)]
        for src, fx in ((ucv, fxu), (vcv, fxv)):
            for j in range(C // 2):
                e0, e1 = 2 * j, 2 * j + 1
                p0 = src.at[jnp.full((L,), e0, jnp.int32)].get(
                    mode="promise_in_bounds")
                p1 = src.at[jnp.full((L,), e1, jnp.int32)].get(
                    mode="promise_in_bounds")
                fx[pl.ds(j * L, L)] = jnp.where(
                    iota < S, p0 * S + iota, p1 * S + (iota - S))

        l1 = [
            (nuu_f, fxu, iuu), (nui_f, fxu, iui),
            (niu_f, fxv, iiu), (nii_f, fxv, iii), (nir_f, fxv, iir),
            (st_f, fxu, est),
            (ratui_f, fxu, eratui), (votui_f, fxu, evotui),
            (timui_f, fxu, etimui),
            (ratiu_f, fxv, eratiu), (votiu_f, fxv, evotiu),
            (timiu_f, fxv, etimiu),
        ]
        for tab, fx, dst in l1:
            pltpu.async_copy(tab.at[fx], dst, sem1)
        pltpu.async_copy(usr_hbm.at[uc], uself_v, sem1)
        pltpu.async_copy(item_hbm.at[vc], vself_v, sem1)
        for tab, fx, dst in l1:
            pltpu.make_async_copy(tab.at[fx], dst, sem1).wait()
        pltpu.make_async_copy(usr_hbm.at[uc], uself_v, sem1).wait()
        pltpu.make_async_copy(item_hbm.at[vc], vself_v, sem1).wait()

        row8 = pl.multiple_of(off * S, C * S)
        outs = [
            (est, st_out), (eratui, ratui_out), (evotui, votui_out),
            (etimui, timui_out), (eratiu, ratiu_out), (evotiu, votiu_out),
            (etimiu, timiu_out),
        ]
        for src, dst in outs:
            pltpu.async_copy(src, dst.at[pl.ds(row8, C * S)], semo)
        pltpu.async_copy(uself_v, uself_out.at[pl.ds(off, C)], semo)
        pltpu.async_copy(vself_v, vself_out.at[pl.ds(off, C)], semo)

        l2 = [
            (usr_hbm, iuu, ruu, uu_out), (item_hbm, iui, rui, ui_out),
            (usr_hbm, iiu, riu, iu_out), (item_hbm, iii, rii, ii_out),
            (rel_hbm, iir, rir, ir_out),
        ]
        for tab, fx, dst, _o in l2:
            pltpu.async_copy(tab.at[fx], dst, sem2)
        for tab, fx, dst, _o in l2:
            pltpu.make_async_copy(tab.at[fx], dst, sem2).wait()

        for _t, _f, dst, out in l2:
            pltpu.async_copy(dst, out.at[pl.ds(row8, C * S)], semo)

        for src, dst in outs:
            pltpu.make_async_copy(src, dst.at[pl.ds(row8, C * S)],
                                  semo).wait()
        pltpu.make_async_copy(uself_v, uself_out.at[pl.ds(off, C)],
                              semo).wait()
        pltpu.make_async_copy(vself_v, vself_out.at[pl.ds(off, C)],
                              semo).wait()
        for _t, _f, dst, out in l2:
            pltpu.make_async_copy(dst, out.at[pl.ds(row8, C * S)],
                                  semo).wait()
        return 0

    lax.fori_loop(0, NCH, chunk_body, 0, unroll=False)


def _sc_gather(u, v, usr_feat, item_feat, rel_feat,
               nuu, nui, niu, nii, nir,
               st, ratui, votui, timui, ratiu, votiu, timiu):
    f32, i32 = jnp.float32, jnp.int32
    row = lambda n: jax.ShapeDtypeStruct((n, D), f32)
    fl = lambda: jax.ShapeDtypeStruct((B * S,), f32)
    kfn = pl.kernel(
        _sc_body,
        out_type=[row(B * S), row(B * S), row(B * S), row(B * S), row(B * S),
                  row(B), row(B),
                  fl(), fl(), fl(), fl(), fl(), fl(), fl()],
        mesh=plsc.VectorSubcoreMesh(core_axis_name="c", subcore_axis_name="s"),
        compiler_params=pltpu.CompilerParams(use_tc_tiling_on_sc=False),
        scratch_types=(
            [pltpu.VMEM((E,), i32) for _ in range(2)]
            + [pltpu.VMEM((C * S,), i32) for _ in range(2)]
            + [pltpu.VMEM((C * S,), i32) for _ in range(5)]
            + [pltpu.VMEM((C * S,), f32) for _ in range(7)]
            + [pltpu.VMEM((C, D), f32) for _ in range(2)]
            + [pltpu.VMEM((C * S, D), f32) for _ in range(5)]
            + [pltpu.SemaphoreType.DMA for _ in range(3)]
        ),
    )
    return kfn(u, v, usr_feat, item_feat, rel_feat,
               nuu, nui, niu, nii, nir,
               st, ratui, votui, timui, ratiu, votiu, timiu)


def _tc_body(uu_ref, ui_ref, iu_ref, ii_ref, ir_ref,
             uself_ref, vself_ref,
             st_ref, ratui_ref, votui_ref, timui_ref,
             ratiu_ref, votiu_ref, timiu_ref,
             wu_ref, bu_ref, wv_ref, bv_ref,
             out_ref):
    uself = uself_ref[...]
    vself = vself_ref[...]

    def wsum(w, rows_ref):
        rows = rows_ref[...]
        acc = w[:, 0:1] * rows[:, 0:D]
        for s in range(1, S):
            acc = acc + w[:, s:s + 1] * rows[:, s * D:(s + 1) * D]
        return acc

    def softmax8(z):
        m = jnp.max(z, axis=1, keepdims=True)
        e = jnp.exp(z - m)
        return e / jnp.sum(e, axis=1, keepdims=True)

    st = jax.nn.sigmoid(st_ref[...])
    uu_agg = wsum(st, uu_ref) * (1.0 / S)
    z_ui = ratui_ref[...] * votui_ref[...] + timui_ref[...]
    ui_agg = wsum(softmax8(z_ui), ui_ref)
    u_vec = jnp.tanh(
        jnp.dot(uself + uu_agg + ui_agg, wu_ref[...],
                preferred_element_type=jnp.float32) + bu_ref[...])

    z_iu = ratiu_ref[...] * votiu_ref[...] + timiu_ref[...]
    iu_agg = wsum(softmax8(z_iu), iu_ref)
    ir = ir_ref[...]
    pi = jnp.concatenate(
        [jnp.sum(ir[:, s * D:(s + 1) * D] * uself, axis=1, keepdims=True)
         for s in range(S)], axis=1)
    ii_agg = wsum(softmax8(pi), ii_ref)
    v_vec = jnp.tanh(
        jnp.dot(vself + iu_agg + ii_agg, wv_ref[...],
                preferred_element_type=jnp.float32) + bv_ref[...])

    sdot = jnp.sum(u_vec * v_vec, axis=1)
    out_ref[...] = 5.0 / (1.0 + jnp.exp(-sdot))


def _tc_math(uu, ui, iu, ii, ir, uself, vself,
             st, ratui, votui, timui, ratiu, votiu, timiu,
             W_u, b_u, W_v, b_v):
    n = B // R
    bs_r = lambda w: pl.BlockSpec((R, w), lambda i: (i, 0))
    bs_w = pl.BlockSpec((D, D), lambda i: (0, 0))
    bs_b = pl.BlockSpec((1, D), lambda i: (0, 0))
    return pl.pallas_call(
        _tc_body,
        grid=(n,),
        in_specs=[bs_r(S * D), bs_r(S * D), bs_r(S * D), bs_r(S * D),
                  bs_r(S * D),
                  bs_r(D), bs_r(D),
                  bs_r(S), bs_r(S), bs_r(S), bs_r(S),
                  bs_r(S), bs_r(S), bs_r(S),
                  bs_w, bs_b, bs_w, bs_b],
        out_specs=pl.BlockSpec((R,), lambda i: (i,)),
        out_shape=jax.ShapeDtypeStruct((B,), jnp.float32),
    )(uu, ui, iu, ii, ir, uself, vself,
      st, ratui, votui, timui, ratiu, votiu, timiu,
      W_u, b_u.reshape(1, D), W_v, b_v.reshape(1, D))


def kernel(u, v, usr_feat, item_feat, rel_feat,
           neigh_uu, neigh_uu_st, neigh_ui, neigh_ui_rat, neigh_ui_vot,
           neigh_ui_tim, neigh_iu, neigh_iu_rat, neigh_iu_vot, neigh_iu_tim,
           neigh_ii, neigh_ir, W_u, b_u, W_v, b_v):
    i32 = jnp.int32
    flat = lambda t: t.reshape(-1)
    flati = lambda t: t.astype(i32).reshape(-1)
    outs = _sc_gather(
        u.astype(i32), v.astype(i32),
        usr_feat, item_feat, rel_feat,
        flati(neigh_uu), flati(neigh_ui), flati(neigh_iu),
        flati(neigh_ii), flati(neigh_ir),
        flat(neigh_uu_st), flat(neigh_ui_rat), flat(neigh_ui_vot),
        flat(neigh_ui_tim), flat(neigh_iu_rat), flat(neigh_iu_vot),
        flat(neigh_iu_tim))
    (uu, ui, iu, ii, ir, uself, vself,
     st, ratui, votui, timui, ratiu, votiu, timiu) = outs
    wide = lambda t: t.reshape(B, S * D)
    nar = lambda t: t.reshape(B, S)
    return _tc_math(
        wide(uu), wide(ui), wide(iu), wide(ii), wide(ir),
        uself, vself,
        nar(st), nar(ratui), nar(votui), nar(timui),
        nar(ratiu), nar(votiu), nar(timiu),
        W_u, b_u.reshape(1, D), W_v, b_v.reshape(1, D))


# speedup vs baseline: 1.0286x; 1.0286x over previous
import jax
import jax.numpy as jnp
from jax import lax
from jax.experimental import pallas as pl
from jax.experimental.pallas import tpu as pltpu
from jax.experimental.pallas import tpu_sc as plsc

NC = 2
NS = 16
L = 16
NW = NC * NS

B = 16384
D = 16
S = 8
E = B // NW
C = 16
NCH = E // C
R = 512


def _build_fx(u_v, v_v, fxu, fxv, loc, iota):
    ucv = u_v[pl.ds(loc, C)]
    vcv = v_v[pl.ds(loc, C)]
    for src, fx in ((ucv, fxu), (vcv, fxv)):
        for j in range(C // 2):
            e0, e1 = 2 * j, 2 * j + 1
            p0 = src.at[jnp.full((L,), e0, jnp.int32)].get(
                mode="promise_in_bounds")
            p1 = src.at[jnp.full((L,), e1, jnp.int32)].get(
                mode="promise_in_bounds")
            fx[pl.ds(j * L, L)] = jnp.where(
                iota < S, p0 * S + iota, p1 * S + (iota - S))


def _sc_body_a(u_hbm, v_hbm,
               usr_hbm, item_hbm, rel_hbm,
               nuu_f, nui_f, niu_f, nii_f, nir_f,
               uu_out, ui_out, iu_out, ii_out, ir_out,
               uself_out, vself_out,
               u_v, v_v, fxu, fxv,
               iuu, iui, iiu, iii, iir,
               uself_v, vself_v,
               ruu, rui, riu, rii, rir,
               sem1, sem2, semo):
    wid = lax.axis_index("s") * NC + lax.axis_index("c")
    base = pl.multiple_of(wid * E, E)
    iota = lax.iota(jnp.int32, L)

    pltpu.sync_copy(u_hbm.at[pl.ds(base, E)], u_v)
    pltpu.sync_copy(v_hbm.at[pl.ds(base, E)], v_v)

    def chunk_body(c, _):
        loc = pl.multiple_of(c * C, C)
        off = pl.multiple_of(base + c * C, C)
        uc = u_v.at[pl.ds(loc, C)]
        vc = v_v.at[pl.ds(loc, C)]
        _build_fx(u_v, v_v, fxu, fxv, loc, iota)

        l1 = [
            (nuu_f, fxu, iuu), (nui_f, fxu, iui),
            (niu_f, fxv, iiu), (nii_f, fxv, iii), (nir_f, fxv, iir),
        ]
        for tab, fx, dst in l1:
            pltpu.async_copy(tab.at[fx], dst, sem1)
        pltpu.async_copy(usr_hbm.at[uc], uself_v, sem1)
        pltpu.async_copy(item_hbm.at[vc], vself_v, sem1)
        for tab, fx, dst in l1:
            pltpu.make_async_copy(tab.at[fx], dst, sem1).wait()
        pltpu.make_async_copy(usr_hbm.at[uc], uself_v, sem1).wait()
        pltpu.make_async_copy(item_hbm.at[vc], vself_v, sem1).wait()

        row8 = pl.multiple_of(off * S, C * S)
        pltpu.async_copy(uself_v, uself_out.at[pl.ds(off, C)], semo)
        pltpu.async_copy(vself_v, vself_out.at[pl.ds(off, C)], semo)

        l2 = [
            (usr_hbm, iuu, ruu, uu_out), (item_hbm, iui, rui, ui_out),
            (usr_hbm, iiu, riu, iu_out), (item_hbm, iii, rii, ii_out),
            (rel_hbm, iir, rir, ir_out),
        ]
        for tab, fx, dst, _o in l2:
            pltpu.async_copy(tab.at[fx], dst, sem2)
        for tab, fx, dst, _o in l2:
            pltpu.make_async_copy(tab.at[fx], dst, sem2).wait()

        for _t, _f, dst, out in l2:
            pltpu.async_copy(dst, out.at[pl.ds(row8, C * S)], semo)

        pltpu.make_async_copy(uself_v, uself_out.at[pl.ds(off, C)],
                              semo).wait()
        pltpu.make_async_copy(vself_v, vself_out.at[pl.ds(off, C)],
                              semo).wait()
        for _t, _f, dst, out in l2:
            pltpu.make_async_copy(dst, out.at[pl.ds(row8, C * S)],
                                  semo).wait()
        return 0

    lax.fori_loop(0, NCH, chunk_body, 0, unroll=False)


def _sc_body_b(u_hbm, v_hbm,
               st_f, ratui_f, votui_f, timui_f,
               ratiu_f, votiu_f, timiu_f,
               st_out, ratui_out, votui_out, timui_out,
               ratiu_out, votiu_out, timiu_out,
               u_v, v_v, fxu, fxv,
               est, eratui, evotui, etimui, eratiu, evotiu, etimiu,
               sem1, semo):
    wid = lax.axis_index("s") * NC + lax.axis_index("c")
    base = pl.multiple_of(wid * E, E)
    iota = lax.iota(jnp.int32, L)

    pltpu.sync_copy(u_hbm.at[pl.ds(base, E)], u_v)
    pltpu.sync_copy(v_hbm.at[pl.ds(base, E)], v_v)

    def chunk_body(c, _):
        loc = pl.multiple_of(c * C, C)
        off = pl.multiple_of(base + c * C, C)
        _build_fx(u_v, v_v, fxu, fxv, loc, iota)

        l1 = [
            (st_f, fxu, est),
            (ratui_f, fxu, eratui), (votui_f, fxu, evotui),
            (timui_f, fxu, etimui),
            (ratiu_f, fxv, eratiu), (votiu_f, fxv, evotiu),
            (timiu_f, fxv, etimiu),
        ]
        for tab, fx, dst in l1:
            pltpu.async_copy(tab.at[fx], dst, sem1)
        for tab, fx, dst in l1:
            pltpu.make_async_copy(tab.at[fx], dst, sem1).wait()

        row8 = pl.multiple_of(off * S, C * S)
        outs = [
            (est, st_out), (eratui, ratui_out), (evotui, votui_out),
            (etimui, timui_out), (eratiu, ratiu_out), (evotiu, votiu_out),
            (etimiu, timiu_out),
        ]
        for src, dst in outs:
            pltpu.async_copy(src, dst.at[pl.ds(row8, C * S)], semo)
        for src, dst in outs:
            pltpu.make_async_copy(src, dst.at[pl.ds(row8, C * S)],
                                  semo).wait()
        return 0

    lax.fori_loop(0, NCH, chunk_body, 0, unroll=False)


def _sc_gather_a(u, v, usr_feat, item_feat, rel_feat,
                 nuu, nui, niu, nii, nir):
    f32, i32 = jnp.float32, jnp.int32
    row = lambda n: jax.ShapeDtypeStruct((n, D), f32)
    kfn = pl.kernel(
        _sc_body_a,
        out_type=[row(B * S), row(B * S), row(B * S), row(B * S), row(B * S),
                  row(B), row(B)],
        mesh=plsc.VectorSubcoreMesh(core_axis_name="c", subcore_axis_name="s"),
        compiler_params=pltpu.CompilerParams(use_tc_tiling_on_sc=False),
        scratch_types=(
            [pltpu.VMEM((E,), i32) for _ in range(2)]
            + [pltpu.VMEM((C * S,), i32) for _ in range(2)]
            + [pltpu.VMEM((C * S,), i32) for _ in range(5)]
            + [pltpu.VMEM((C, D), f32) for _ in range(2)]
            + [pltpu.VMEM((C * S, D), f32) for _ in range(5)]
            + [pltpu.SemaphoreType.DMA for _ in range(3)]
        ),
    )
    return kfn(u, v, usr_feat, item_feat, rel_feat, nuu, nui, niu, nii, nir)


def _sc_gather_b(u, v, st, ratui, votui, timui, ratiu, votiu, timiu):
    f32, i32 = jnp.float32, jnp.int32
    fl = lambda: jax.ShapeDtypeStruct((B * S,), f32)
    kfn = pl.kernel(
        _sc_body_b,
        out_type=[fl(), fl(), fl(), fl(), fl(), fl(), fl()],
        mesh=plsc.VectorSubcoreMesh(core_axis_name="c", subcore_axis_name="s"),
        compiler_params=pltpu.CompilerParams(use_tc_tiling_on_sc=False),
        scratch_types=(
            [pltpu.VMEM((E,), i32) for _ in range(2)]
            + [pltpu.VMEM((C * S,), i32) for _ in range(2)]
            + [pltpu.VMEM((C * S,), f32) for _ in range(7)]
            + [pltpu.SemaphoreType.DMA for _ in range(2)]
        ),
    )
    return kfn(u, v, st, ratui, votui, timui, ratiu, votiu, timiu)


def _tc_body(uu_ref, ui_ref, iu_ref, ii_ref, ir_ref,
             uself_ref, vself_ref,
             st_ref, ratui_ref, votui_ref, timui_ref,
             ratiu_ref, votiu_ref, timiu_ref,
             wu_ref, bu_ref, wv_ref, bv_ref,
             out_ref):
    uself = uself_ref[...]
    vself = vself_ref[...]

    def wsum(w, rows_ref):
        rows = rows_ref[...]
        acc = w[:, 0:1] * rows[:, 0:D]
        for s in range(1, S):
            acc = acc + w[:, s:s + 1] * rows[:, s * D:(s + 1) * D]
        return acc

    def softmax8(z):
        m = jnp.max(z, axis=1, keepdims=True)
        e = jnp.exp(z - m)
        return e / jnp.sum(e, axis=1, keepdims=True)

    st = jax.nn.sigmoid(st_ref[...])
    uu_agg = wsum(st, uu_ref) * (1.0 / S)
    z_ui = ratui_ref[...] * votui_ref[...] + timui_ref[...]
    ui_agg = wsum(softmax8(z_ui), ui_ref)
    u_vec = jnp.tanh(
        jnp.dot(uself + uu_agg + ui_agg, wu_ref[...],
                preferred_element_type=jnp.float32) + bu_ref[...])

    z_iu = ratiu_ref[...] * votiu_ref[...] + timiu_ref[...]
    iu_agg = wsum(softmax8(z_iu), iu_ref)
    ir = ir_ref[...]
    pi = jnp.concatenate(
        [jnp.sum(ir[:, s * D:(s + 1) * D] * uself, axis=1, keepdims=True)
         for s in range(S)], axis=1)
    ii_agg = wsum(softmax8(pi), ii_ref)
    v_vec = jnp.tanh(
        jnp.dot(vself + iu_agg + ii_agg, wv_ref[...],
                preferred_element_type=jnp.float32) + bv_ref[...])

    sdot = jnp.sum(u_vec * v_vec, axis=1)
    out_ref[...] = 5.0 / (1.0 + jnp.exp(-sdot))


def _tc_math(uu, ui, iu, ii, ir, uself, vself,
             st, ratui, votui, timui, ratiu, votiu, timiu,
             W_u, b_u, W_v, b_v):
    n = B // R
    bs_r = lambda w: pl.BlockSpec((R, w), lambda i: (i, 0))
    bs_w = pl.BlockSpec((D, D), lambda i: (0, 0))
    bs_b = pl.BlockSpec((1, D), lambda i: (0, 0))
    return pl.pallas_call(
        _tc_body,
        grid=(n,),
        in_specs=[bs_r(S * D), bs_r(S * D), bs_r(S * D), bs_r(S * D),
                  bs_r(S * D),
                  bs_r(D), bs_r(D),
                  bs_r(S), bs_r(S), bs_r(S), bs_r(S),
                  bs_r(S), bs_r(S), bs_r(S),
                  bs_w, bs_b, bs_w, bs_b],
        out_specs=pl.BlockSpec((R,), lambda i: (i,)),
        out_shape=jax.ShapeDtypeStruct((B,), jnp.float32),
    )(uu, ui, iu, ii, ir, uself, vself,
      st, ratui, votui, timui, ratiu, votiu, timiu,
      W_u, b_u.reshape(1, D), W_v, b_v.reshape(1, D))


def kernel(u, v, usr_feat, item_feat, rel_feat,
           neigh_uu, neigh_uu_st, neigh_ui, neigh_ui_rat, neigh_ui_vot,
           neigh_ui_tim, neigh_iu, neigh_iu_rat, neigh_iu_vot, neigh_iu_tim,
           neigh_ii, neigh_ir, W_u, b_u, W_v, b_v):
    i32 = jnp.int32
    flat = lambda t: t.reshape(-1)
    flati = lambda t: t.astype(i32).reshape(-1)
    ui32 = u.astype(i32)
    vi32 = v.astype(i32)
    (uu, ui, iu, ii, ir, uself, vself) = _sc_gather_a(
        ui32, vi32, usr_feat, item_feat, rel_feat,
        flati(neigh_uu), flati(neigh_ui), flati(neigh_iu),
        flati(neigh_ii), flati(neigh_ir))
    (st, ratui, votui, timui, ratiu, votiu, timiu) = _sc_gather_b(
        ui32, vi32,
        flat(neigh_uu_st), flat(neigh_ui_rat), flat(neigh_ui_vot),
        flat(neigh_ui_tim), flat(neigh_iu_rat), flat(neigh_iu_vot),
        flat(neigh_iu_tim))
    wide = lambda t: t.reshape(B, S * D)
    nar = lambda t: t.reshape(B, S)
    return _tc_math(
        wide(uu), wide(ui), wide(iu), wide(ii), wide(ir),
        uself, vself,
        nar(st), nar(ratui), nar(votui), nar(timui),
        nar(ratiu), nar(votiu), nar(timiu),
        W_u, b_u.reshape(1, D), W_v, b_v.reshape(1, D))
